# baseline (speedup 1.0000x reference)
import jax
import jax.numpy as jnp
from jax import lax
from jax.experimental import pallas as pl
from jax.experimental.pallas import tpu as pltpu
from jax.experimental.pallas import tpu_sc as plsc

A = 32736
AP = 32768
M = 100
MP = 128
B = 8
NC = 2
NS = 16
L = 16
CH = AP // NS
G = CH // L
MT = 104
T = 2048
NT = AP // T
IOU_T = 0.3
BIG = 1 << 30

f32 = jnp.float32
i32 = jnp.int32


def _tc_body(an_ref, bb_ref, rm_ref, ra_ref, cm_ref, ca_ref):
    k = pl.program_id(1)
    tcol = lax.broadcasted_iota(i32, (MT, T), 0)
    aid = lax.broadcasted_iota(i32, (MT, T), 1) + k * T

    acx = an_ref[0, :]
    acy = an_ref[1, :]
    aw = an_ref[2, :]
    ah = an_ref[3, :]
    ax1 = acx - aw / 2.0
    ay1 = acy - ah / 2.0
    ax2 = acx + aw / 2.0
    ay2 = acy + ah / 2.0
    area_a = (ax2 - ax1) * (ay2 - ay1)

    bx1 = bb_ref[0, 0, :, :]
    by1 = bb_ref[0, 1, :, :]
    bx2 = bb_ref[0, 2, :, :]
    by2 = bb_ref[0, 3, :, :]
    area_b = (bx2 - bx1) * (by2 - by1)

    def row(v):
        return jnp.broadcast_to(v[None, :], (MT, T))

    def col(v):
        return jnp.broadcast_to(v, (MT, T))

    ltx = jnp.maximum(row(ax1), col(bx1))
    lty = jnp.maximum(row(ay1), col(by1))
    rbx = jnp.minimum(row(ax2), col(bx2))
    rby = jnp.minimum(row(ay2), col(by2))
    ww = jnp.maximum(rbx - ltx, 0.0)
    hh = jnp.maximum(rby - lty, 0.0)
    inter = ww * hh
    uni = (row(area_a) + col(area_b)) - inter
    iou = inter / uni
    iou = jnp.where(tcol < M, iou, -1.0)

    rm = jnp.max(iou, axis=0)
    ra = jnp.min(jnp.where(iou == rm[None, :], tcol, BIG), axis=0)
    rm_ref[0, 0, 0, :] = rm
    ra_ref[0, 0, 0, :] = ra

    cmt = jnp.max(iou, axis=1, keepdims=True)
    cat = jnp.min(jnp.where(iou == cmt, aid, BIG), axis=1, keepdims=True)
    prev_cm = jnp.where(k == 0, -2.0, cm_ref[0, :, :])
    prev_ca = jnp.where(k == 0, BIG, ca_ref[0, :, :])
    upd = cmt > prev_cm
    eq = cmt == prev_cm
    cm_ref[0, :, :] = jnp.maximum(prev_cm, cmt)
    ca_ref[0, :, :] = jnp.where(upd, cat,
                                jnp.where(eq, jnp.minimum(prev_ca, cat),
                                          prev_ca))


def _sc_body(rm_hbm, ra_hbm, cm_hbm, ca_hbm, bb_hbm, lab_hbm,
             sc_hbm, obb_hbm,
             r_iou, r_arg, bx1, by1, bx2, by2, labv,
             m_iou, m_den, m_id, scb, bb2, sem):
    c = lax.axis_index("c")
    s = lax.axis_index("s")
    base = s * CH
    iota = lax.iota(i32, L)

    def img_body(i, _):
        b = c * (B // NC) + i
        cps = [
            pltpu.async_copy(rm_hbm.at[b, pl.ds(base, CH)], r_iou, sem),
            pltpu.async_copy(ra_hbm.at[b, pl.ds(base, CH)], r_arg, sem),
            pltpu.async_copy(cm_hbm.at[b, :], m_iou, sem),
            pltpu.async_copy(ca_hbm.at[b, :], m_id, sem),
            pltpu.async_copy(bb_hbm.at[b, 0, :], bx1, sem),
            pltpu.async_copy(bb_hbm.at[b, 1, :], by1, sem),
            pltpu.async_copy(bb_hbm.at[b, 2, :], bx2, sem),
            pltpu.async_copy(bb_hbm.at[b, 3, :], by2, sem),
            pltpu.async_copy(lab_hbm.at[b, :], labv, sem),
        ]
        for cp in cps:
            cp.wait()

        def den_body(tg, _):
            sl = pl.ds(tg * L, L)
            m_den[sl] = jnp.maximum(m_iou[sl], IOU_T)
            return 0

        lax.fori_loop(0, MP // L, den_body, 0)

        def ow_body(t, _):
            tsl16 = pl.ds(t, L)
            ai = m_id[tsl16][0]
            off = ai - base
            inr = (off >= 0) & (off < CH)
            offc = jnp.clip(off, 0, CH - 1)
            mk = lax.bitwise_and(iota == 0, jnp.broadcast_to(inr, (L,)))
            iv = jnp.broadcast_to(offc, (L,)).astype(i32)
            plsc.store_scatter(r_iou, [iv],
                               jnp.broadcast_to(m_iou[tsl16][0], (L,)),
                               mask=mk)
            plsc.store_scatter(r_arg, [iv],
                               jnp.broadcast_to(t, (L,)).astype(i32), mask=mk)
            return 0

        lax.fori_loop(0, M, ow_body, 0)

        def fin_body(g, _):
            gs = g * L
            sl = pl.ds(gs, L)
            mi = r_iou[sl]
            bidx = r_arg[sl]
            den = plsc.load_gather(m_den, [bidx])
            lv = plsc.load_gather(labv, [bidx])
            mia = jnp.where(mi < IOU_T * 0.5, 0.0, mi)
            scv = mia / den
            scv = jnp.where(lv <= 0, jnp.zeros((L,), f32), scv)
            scb[sl] = scv
            rows4 = (jnp.broadcast_to(gs, (L,)).astype(i32) + iota) * 4
            for j, ref in enumerate((bx1, by1, bx2, by2)):
                bbj = plsc.load_gather(ref, [bidx])
                plsc.store_scatter(bb2, [rows4 + j], bbj)
            return 0

        lax.fori_loop(0, G, fin_body, 0)

        o1 = pltpu.async_copy(scb, sc_hbm.at[b, pl.ds(base, CH)], sem)
        o2 = pltpu.async_copy(bb2, obb_hbm.at[b, pl.ds(base * 4, CH * 4)],
                              sem)
        o1.wait()
        o2.wait()
        return 0

    lax.fori_loop(0, B // NC, img_body, 0)


@jax.jit
def kernel(labels, bboxes, anchors):
    labp = jnp.pad(labels.astype(i32), ((0, 0), (0, MP - M)))
    bb_soa = jnp.transpose(jnp.pad(bboxes, ((0, 0), (0, MP - M), (0, 0))),
                           (0, 2, 1))
    bbT = bb_soa[:, :, :MT, None]
    anp = jnp.transpose(jnp.pad(anchors, ((0, AP - A), (0, 0))), (1, 0))

    rm, ra, cm, ca = pl.pallas_call(
        _tc_body,
        grid=(B, NT),
        in_specs=[
            pl.BlockSpec((4, T), lambda b, k: (0, k)),
            pl.BlockSpec((1, 4, MT, 1), lambda b, k: (b, 0, 0, 0)),
        ],
        out_specs=[
            pl.BlockSpec((1, 1, 1, T), lambda b, k: (b, k, 0, 0)),
            pl.BlockSpec((1, 1, 1, T), lambda b, k: (b, k, 0, 0)),
            pl.BlockSpec((1, MT, 1), lambda b, k: (b, 0, 0)),
            pl.BlockSpec((1, MT, 1), lambda b, k: (b, 0, 0)),
        ],
        out_shape=[
            jax.ShapeDtypeStruct((B, NT, 1, T), f32),
            jax.ShapeDtypeStruct((B, NT, 1, T), i32),
            jax.ShapeDtypeStruct((B, MT, 1), f32),
            jax.ShapeDtypeStruct((B, MT, 1), i32),
        ],
    )(anp, bbT)

    rm = rm.reshape(B, AP)
    ra = ra.reshape(B, AP)
    cm = jnp.pad(cm.reshape(B, MT), ((0, 0), (0, MP - MT)))
    ca = jnp.pad(ca.reshape(B, MT), ((0, 0), (0, MP - MT)))

    mesh = plsc.VectorSubcoreMesh(core_axis_name="c", subcore_axis_name="s",
                                  num_cores=NC, num_subcores=NS)
    run = pl.kernel(
        _sc_body,
        out_type=[jax.ShapeDtypeStruct((B, AP), f32),
                  jax.ShapeDtypeStruct((B, AP * 4), f32)],
        mesh=mesh,
        compiler_params=pltpu.CompilerParams(needs_layout_passes=False),
        scratch_types=[
            pltpu.VMEM((CH,), f32),
            pltpu.VMEM((CH,), i32),
            pltpu.VMEM((MP,), f32),
            pltpu.VMEM((MP,), f32),
            pltpu.VMEM((MP,), f32),
            pltpu.VMEM((MP,), f32),
            pltpu.VMEM((MP,), i32),
            pltpu.VMEM((MP,), f32),
            pltpu.VMEM((MP,), f32),
            pltpu.VMEM((MP,), i32),
            pltpu.VMEM((CH,), f32),
            pltpu.VMEM((CH * 4,), f32),
            pltpu.SemaphoreType.DMA,
        ],
    )
    scores, obb = run(rm, ra, cm, ca, bb_soa, labp)
    return scores[:, :A], obb.reshape(B, AP, 4)[:, :A, :]


# speedup vs baseline: 1.0348x; 1.0348x over previous
import jax
import jax.numpy as jnp
from jax import lax
from jax.experimental import pallas as pl
from jax.experimental.pallas import tpu as pltpu
from jax.experimental.pallas import tpu_sc as plsc

A = 32736
AP = 32768
M = 100
MP = 128
B = 8
NC = 2
NS = 16
L = 16
CH = AP // NS
G = CH // L
MT = 104
T = 4096
NT = AP // T
IOU_T = 0.3
BIG = 1 << 30

f32 = jnp.float32
i32 = jnp.int32


def _tc_body(an_ref, bb_ref, rm_ref, ra_ref, cm_ref, ca_ref):
    k = pl.program_id(1)
    tcol = lax.broadcasted_iota(i32, (MT, T), 0)
    aid = lax.broadcasted_iota(i32, (MT, T), 1) + k * T

    acx = an_ref[0, :]
    acy = an_ref[1, :]
    aw = an_ref[2, :]
    ah = an_ref[3, :]
    ax1 = acx - aw / 2.0
    ay1 = acy - ah / 2.0
    ax2 = acx + aw / 2.0
    ay2 = acy + ah / 2.0
    area_a = (ax2 - ax1) * (ay2 - ay1)

    bx1 = bb_ref[0, 0, :, :]
    by1 = bb_ref[0, 1, :, :]
    bx2 = bb_ref[0, 2, :, :]
    by2 = bb_ref[0, 3, :, :]
    area_b = (bx2 - bx1) * (by2 - by1)

    def row(v):
        return jnp.broadcast_to(v[None, :], (MT, T))

    def col(v):
        return jnp.broadcast_to(v, (MT, T))

    ltx = jnp.maximum(row(ax1), col(bx1))
    lty = jnp.maximum(row(ay1), col(by1))
    rbx = jnp.minimum(row(ax2), col(bx2))
    rby = jnp.minimum(row(ay2), col(by2))
    ww = jnp.maximum(rbx - ltx, 0.0)
    hh = jnp.maximum(rby - lty, 0.0)
    inter = ww * hh
    uni = (row(area_a) + col(area_b)) - inter
    iou = inter / uni
    iou = jnp.where(tcol < M, iou, -1.0)

    rm = jnp.max(iou, axis=0)
    ra = jnp.min(jnp.where(iou == rm[None, :], tcol, BIG), axis=0)
    rm_ref[0, 0, 0, :] = rm
    ra_ref[0, 0, 0, :] = ra

    cmt = jnp.max(iou, axis=1, keepdims=True)
    cat = jnp.min(jnp.where(iou == cmt, aid, BIG), axis=1, keepdims=True)
    cm_ref[0, 0, :, :] = cmt
    ca_ref[0, 0, :, :] = cat


def _sc_body(rm_hbm, ra_hbm, cm_hbm, ca_hbm, bb_hbm, lab_hbm,
             sc_hbm, obb_hbm,
             r_iou, r_arg, bx1, by1, bx2, by2, labv,
             cpart, capart, m_iou, m_den, m_id, scb, bb2, sem):
    c = lax.axis_index("c")
    s = lax.axis_index("s")
    base = s * CH
    iota = lax.iota(i32, L)

    def img_body(i, _):
        b = c * (B // NC) + i
        cps = [
            pltpu.async_copy(rm_hbm.at[b, pl.ds(base, CH)], r_iou, sem),
            pltpu.async_copy(ra_hbm.at[b, pl.ds(base, CH)], r_arg, sem),
            pltpu.async_copy(cm_hbm.at[b, :], cpart, sem),
            pltpu.async_copy(ca_hbm.at[b, :], capart, sem),
            pltpu.async_copy(bb_hbm.at[b, 0, :], bx1, sem),
            pltpu.async_copy(bb_hbm.at[b, 1, :], by1, sem),
            pltpu.async_copy(bb_hbm.at[b, 2, :], bx2, sem),
            pltpu.async_copy(bb_hbm.at[b, 3, :], by2, sem),
            pltpu.async_copy(lab_hbm.at[b, :], labv, sem),
        ]
        for cp in cps:
            cp.wait()

        def den_body(tg, _):
            sl = pl.ds(tg * L, L)

            def k_body(k2, carry):
                mi, mid = carry
                psl = pl.ds(k2 * MT + tg * L, L)
                pi = cpart[psl]
                pid = capart[psl]
                bt = (pi > mi) | ((pi == mi) & (pid < mid))
                return (jnp.where(bt, pi, mi), jnp.where(bt, pid, mid))

            mi, mid = lax.fori_loop(
                0, NT, k_body,
                (jnp.full((L,), -2.0, f32), jnp.full((L,), BIG, i32)))
            m_iou[sl] = mi
            m_id[sl] = mid
            m_den[sl] = jnp.maximum(mi, IOU_T)
            return 0

        lax.fori_loop(0, (M + L - 1) // L, den_body, 0)

        def ow_body(t, _):
            tsl16 = pl.ds(t, L)
            ai = m_id[tsl16][0]
            off = ai - base
            inr = (off >= 0) & (off < CH)
            offc = jnp.clip(off, 0, CH - 1)
            mk = lax.bitwise_and(iota == 0, jnp.broadcast_to(inr, (L,)))
            iv = jnp.broadcast_to(offc, (L,)).astype(i32)
            plsc.store_scatter(r_iou, [iv],
                               jnp.broadcast_to(m_iou[tsl16][0], (L,)),
                               mask=mk)
            plsc.store_scatter(r_arg, [iv],
                               jnp.broadcast_to(t, (L,)).astype(i32), mask=mk)
            return 0

        lax.fori_loop(0, M, ow_body, 0)

        def fin_body(g, _):
            gs = g * L
            sl = pl.ds(gs, L)
            mi = r_iou[sl]
            bidx = r_arg[sl]
            den = plsc.load_gather(m_den, [bidx])
            lv = plsc.load_gather(labv, [bidx])
            mia = jnp.where(mi < IOU_T * 0.5, 0.0, mi)
            scv = mia / den
            scv = jnp.where(lv <= 0, jnp.zeros((L,), f32), scv)
            scb[sl] = scv
            rows4 = (jnp.broadcast_to(gs, (L,)).astype(i32) + iota) * 4
            for j, ref in enumerate((bx1, by1, bx2, by2)):
                bbj = plsc.load_gather(ref, [bidx])
                plsc.store_scatter(bb2, [rows4 + j], bbj)
            return 0

        lax.fori_loop(0, G, fin_body, 0)

        o1 = pltpu.async_copy(scb, sc_hbm.at[b, pl.ds(base, CH)], sem)
        o2 = pltpu.async_copy(bb2, obb_hbm.at[b, pl.ds(base * 4, CH * 4)],
                              sem)
        o1.wait()
        o2.wait()
        return 0

    lax.fori_loop(0, B // NC, img_body, 0)


@jax.jit
def kernel(labels, bboxes, anchors):
    labp = jnp.pad(labels.astype(i32), ((0, 0), (0, MP - M)))
    bb_soa = jnp.transpose(jnp.pad(bboxes, ((0, 0), (0, MP - M), (0, 0))),
                           (0, 2, 1))
    bbT = bb_soa[:, :, :MT, None]
    anp = jnp.transpose(jnp.pad(anchors, ((0, AP - A), (0, 0))), (1, 0))

    rm, ra, cm, ca = pl.pallas_call(
        _tc_body,
        grid=(B, NT),
        in_specs=[
            pl.BlockSpec((4, T), lambda b, k: (0, k)),
            pl.BlockSpec((1, 4, MT, 1), lambda b, k: (b, 0, 0, 0)),
        ],
        out_specs=[
            pl.BlockSpec((1, 1, 1, T), lambda b, k: (b, k, 0, 0)),
            pl.BlockSpec((1, 1, 1, T), lambda b, k: (b, k, 0, 0)),
            pl.BlockSpec((1, 1, MT, 1), lambda b, k: (b, k, 0, 0)),
            pl.BlockSpec((1, 1, MT, 1), lambda b, k: (b, k, 0, 0)),
        ],
        out_shape=[
            jax.ShapeDtypeStruct((B, NT, 1, T), f32),
            jax.ShapeDtypeStruct((B, NT, 1, T), i32),
            jax.ShapeDtypeStruct((B, NT, MT, 1), f32),
            jax.ShapeDtypeStruct((B, NT, MT, 1), i32),
        ],
    )(anp, bbT)

    rm = rm.reshape(B, AP)
    ra = ra.reshape(B, AP)
    cm = jnp.pad(cm.reshape(B, NT * MT), ((0, 0), (0, 896 - NT * MT)))
    ca = jnp.pad(ca.reshape(B, NT * MT), ((0, 0), (0, 896 - NT * MT)))

    mesh = plsc.VectorSubcoreMesh(core_axis_name="c", subcore_axis_name="s",
                                  num_cores=NC, num_subcores=NS)
    run = pl.kernel(
        _sc_body,
        out_type=[jax.ShapeDtypeStruct((B, AP), f32),
                  jax.ShapeDtypeStruct((B, AP * 4), f32)],
        mesh=mesh,
        compiler_params=pltpu.CompilerParams(needs_layout_passes=False),
        scratch_types=[
            pltpu.VMEM((CH,), f32),
            pltpu.VMEM((CH,), i32),
            pltpu.VMEM((MP,), f32),
            pltpu.VMEM((MP,), f32),
            pltpu.VMEM((MP,), f32),
            pltpu.VMEM((MP,), f32),
            pltpu.VMEM((MP,), i32),
            pltpu.VMEM((896,), f32),
            pltpu.VMEM((896,), i32),
            pltpu.VMEM((MP,), f32),
            pltpu.VMEM((MP,), f32),
            pltpu.VMEM((MP,), i32),
            pltpu.VMEM((CH,), f32),
            pltpu.VMEM((CH * 4,), f32),
            pltpu.SemaphoreType.DMA,
        ],
    )
    scores, obb = run(rm, ra, cm, ca, bb_soa, labp)
    return scores[:, :A], obb.reshape(B, AP, 4)[:, :A, :]


# speedup vs baseline: 1.0456x; 1.0105x over previous
import jax
import jax.numpy as jnp
from jax import lax
from jax.experimental import pallas as pl
from jax.experimental.pallas import tpu as pltpu
from jax.experimental.pallas import tpu_sc as plsc

A = 32736
AP = 32768
M = 100
MP = 128
B = 8
NC = 2
NS = 16
L = 16
CH = AP // NS
G = CH // L
MT = 104
T = 4096
NT = AP // T
IOU_T = 0.3
BIG = 1 << 30

f32 = jnp.float32
i32 = jnp.int32


def _tc_body(an_ref, bb_ref, rm_ref, ra_ref, cm_ref, ca_ref):
    k = pl.program_id(1)
    tcol = lax.broadcasted_iota(i32, (MT, T), 0)
    aid = lax.broadcasted_iota(i32, (MT, T), 1) + k * T

    acx = an_ref[0, :]
    acy = an_ref[1, :]
    aw = an_ref[2, :]
    ah = an_ref[3, :]
    ax1 = acx - aw / 2.0
    ay1 = acy - ah / 2.0
    ax2 = acx + aw / 2.0
    ay2 = acy + ah / 2.0
    area_a = (ax2 - ax1) * (ay2 - ay1)

    bx1 = bb_ref[0, 0, :, :]
    by1 = bb_ref[0, 1, :, :]
    bx2 = bb_ref[0, 2, :, :]
    by2 = bb_ref[0, 3, :, :]
    area_b = (bx2 - bx1) * (by2 - by1)

    def row(v):
        return jnp.broadcast_to(v[None, :], (MT, T))

    def col(v):
        return jnp.broadcast_to(v, (MT, T))

    ltx = jnp.maximum(row(ax1), col(bx1))
    lty = jnp.maximum(row(ay1), col(by1))
    rbx = jnp.minimum(row(ax2), col(bx2))
    rby = jnp.minimum(row(ay2), col(by2))
    ww = jnp.maximum(rbx - ltx, 0.0)
    hh = jnp.maximum(rby - lty, 0.0)
    inter = ww * hh
    uni = (row(area_a) + col(area_b)) - inter
    iou = inter / uni
    iou = jnp.where(tcol < M, iou, -1.0)

    rm = jnp.max(iou, axis=0)
    ra = jnp.min(jnp.where(iou == rm[None, :], tcol, BIG), axis=0)
    rm_ref[0, 0, 0, :] = rm
    ra_ref[0, 0, 0, :] = ra

    cmt = jnp.max(iou, axis=1, keepdims=True)
    cat = jnp.min(jnp.where(iou == cmt, aid, BIG), axis=1, keepdims=True)
    prev_cm = jnp.where(k == 0, -2.0, cm_ref[0, :, :])
    prev_ca = jnp.where(k == 0, BIG, ca_ref[0, :, :])
    upd = cmt > prev_cm
    eq = cmt == prev_cm
    cm_ref[0, :, :] = jnp.maximum(prev_cm, cmt)
    ca_ref[0, :, :] = jnp.where(upd, cat,
                                jnp.where(eq, jnp.minimum(prev_ca, cat),
                                          prev_ca))


def _sc_body(rm_hbm, ra_hbm, cm_hbm, ca_hbm, bb_hbm, lab_hbm,
             sc_hbm, obb_hbm,
             r_iou, r_arg, bx1, by1, bx2, by2, labv,
             m_iou, m_den, m_id, scb, bb2, sem):
    c = lax.axis_index("c")
    s = lax.axis_index("s")
    base = s * CH
    iota = lax.iota(i32, L)

    def img_body(i, _):
        b = c * (B // NC) + i
        cps = [
            pltpu.async_copy(rm_hbm.at[b, pl.ds(base, CH)], r_iou, sem),
            pltpu.async_copy(ra_hbm.at[b, pl.ds(base, CH)], r_arg, sem),
            pltpu.async_copy(cm_hbm.at[b, :], m_iou, sem),
            pltpu.async_copy(ca_hbm.at[b, :], m_id, sem),
            pltpu.async_copy(bb_hbm.at[b, 0, :], bx1, sem),
            pltpu.async_copy(bb_hbm.at[b, 1, :], by1, sem),
            pltpu.async_copy(bb_hbm.at[b, 2, :], bx2, sem),
            pltpu.async_copy(bb_hbm.at[b, 3, :], by2, sem),
            pltpu.async_copy(lab_hbm.at[b, :], labv, sem),
        ]
        for cp in cps:
            cp.wait()

        def den_body(tg, _):
            sl = pl.ds(tg * L, L)
            m_den[sl] = jnp.maximum(m_iou[sl], IOU_T)
            return 0

        lax.fori_loop(0, MP // L, den_body, 0)

        def ow_body(t, _):
            tsl16 = pl.ds(t, L)
            ai = m_id[tsl16][0]
            off = ai - base
            inr = (off >= 0) & (off < CH)
            offc = jnp.clip(off, 0, CH - 1)
            mk = lax.bitwise_and(iota == 0, jnp.broadcast_to(inr, (L,)))
            iv = jnp.broadcast_to(offc, (L,)).astype(i32)
            plsc.store_scatter(r_iou, [iv],
                               jnp.broadcast_to(m_iou[tsl16][0], (L,)),
                               mask=mk)
            plsc.store_scatter(r_arg, [iv],
                               jnp.broadcast_to(t, (L,)).astype(i32), mask=mk)
            return 0

        lax.fori_loop(0, M, ow_body, 0)

        def fin_body(g, _):
            gs = g * L
            sl = pl.ds(gs, L)
            mi = r_iou[sl]
            bidx = r_arg[sl]
            den = plsc.load_gather(m_den, [bidx])
            lv = plsc.load_gather(labv, [bidx])
            mia = jnp.where(mi < IOU_T * 0.5, 0.0, mi)
            scv = mia / den
            scv = jnp.where(lv <= 0, jnp.zeros((L,), f32), scv)
            scb[sl] = scv
            rows4 = (jnp.broadcast_to(gs, (L,)).astype(i32) + iota) * 4
            for j, ref in enumerate((bx1, by1, bx2, by2)):
                bbj = plsc.load_gather(ref, [bidx])
                plsc.store_scatter(bb2, [rows4 + j], bbj)
            return 0

        lax.fori_loop(0, G, fin_body, 0)

        o1 = pltpu.async_copy(scb, sc_hbm.at[b, pl.ds(base, CH)], sem)
        o2 = pltpu.async_copy(bb2, obb_hbm.at[b, pl.ds(base * 4, CH * 4)],
                              sem)
        o1.wait()
        o2.wait()
        return 0

    lax.fori_loop(0, B // NC, img_body, 0)


@jax.jit
def kernel(labels, bboxes, anchors):
    labp = jnp.pad(labels.astype(i32), ((0, 0), (0, MP - M)))
    bb_soa = jnp.transpose(jnp.pad(bboxes, ((0, 0), (0, MP - M), (0, 0))),
                           (0, 2, 1))
    bbT = bb_soa[:, :, :MT, None]
    anp = jnp.transpose(jnp.pad(anchors, ((0, AP - A), (0, 0))), (1, 0))

    rm, ra, cm, ca = pl.pallas_call(
        _tc_body,
        grid=(B, NT),
        in_specs=[
            pl.BlockSpec((4, T), lambda b, k: (0, k)),
            pl.BlockSpec((1, 4, MT, 1), lambda b, k: (b, 0, 0, 0)),
        ],
        out_specs=[
            pl.BlockSpec((1, 1, 1, T), lambda b, k: (b, k, 0, 0)),
            pl.BlockSpec((1, 1, 1, T), lambda b, k: (b, k, 0, 0)),
            pl.BlockSpec((1, MT, 1), lambda b, k: (b, 0, 0)),
            pl.BlockSpec((1, MT, 1), lambda b, k: (b, 0, 0)),
        ],
        out_shape=[
            jax.ShapeDtypeStruct((B, NT, 1, T), f32),
            jax.ShapeDtypeStruct((B, NT, 1, T), i32),
            jax.ShapeDtypeStruct((B, MT, 1), f32),
            jax.ShapeDtypeStruct((B, MT, 1), i32),
        ],
    )(anp, bbT)

    rm = rm.reshape(B, AP)
    ra = ra.reshape(B, AP)
    cm = jnp.pad(cm.reshape(B, MT), ((0, 0), (0, MP - MT)))
    ca = jnp.pad(ca.reshape(B, MT), ((0, 0), (0, MP - MT)))

    mesh = plsc.VectorSubcoreMesh(core_axis_name="c", subcore_axis_name="s",
                                  num_cores=NC, num_subcores=NS)
    run = pl.kernel(
        _sc_body,
        out_type=[jax.ShapeDtypeStruct((B, AP), f32),
                  jax.ShapeDtypeStruct((B, AP * 4), f32)],
        mesh=mesh,
        compiler_params=pltpu.CompilerParams(needs_layout_passes=False),
        scratch_types=[
            pltpu.VMEM((CH,), f32),
            pltpu.VMEM((CH,), i32),
            pltpu.VMEM((MP,), f32),
            pltpu.VMEM((MP,), f32),
            pltpu.VMEM((MP,), f32),
            pltpu.VMEM((MP,), f32),
            pltpu.VMEM((MP,), i32),
            pltpu.VMEM((MP,), f32),
            pltpu.VMEM((MP,), f32),
            pltpu.VMEM((MP,), i32),
            pltpu.VMEM((CH,), f32),
            pltpu.VMEM((CH * 4,), f32),
            pltpu.SemaphoreType.DMA,
        ],
    )
    scores, obb = run(rm, ra, cm, ca, bb_soa, labp)
    return scores[:, :A], obb.reshape(B, AP, 4)[:, :A, :]


# speedup vs baseline: 1.0592x; 1.0130x over previous
import jax
import jax.numpy as jnp
from jax import lax
from jax.experimental import pallas as pl
from jax.experimental.pallas import tpu as pltpu
from jax.experimental.pallas import tpu_sc as plsc

A = 32736
AP = 32768
M = 100
MP = 128
B = 8
NC = 2
NS = 16
L = 16
CH = AP // NS
G = CH // L
MT = 104
T = 4096
NT = AP // T
IOU_T = 0.3
BIG = 1 << 30

f32 = jnp.float32
i32 = jnp.int32


def _tc_body(an_ref, bb_ref, rm_ref, ra_ref, cm_ref, ca_ref):
    k = pl.program_id(1)
    tcol = lax.broadcasted_iota(i32, (MT, T), 0)
    aid = lax.broadcasted_iota(i32, (MT, T), 1) + k * T

    acx = an_ref[0, :]
    acy = an_ref[1, :]
    aw = an_ref[2, :]
    ah = an_ref[3, :]
    ax1 = acx - aw / 2.0
    ay1 = acy - ah / 2.0
    ax2 = acx + aw / 2.0
    ay2 = acy + ah / 2.0
    area_a = (ax2 - ax1) * (ay2 - ay1)

    bx1 = bb_ref[0, 0, :, :]
    by1 = bb_ref[0, 1, :, :]
    bx2 = bb_ref[0, 2, :, :]
    by2 = bb_ref[0, 3, :, :]
    area_b = (bx2 - bx1) * (by2 - by1)

    def row(v):
        return jnp.broadcast_to(v[None, :], (MT, T))

    def col(v):
        return jnp.broadcast_to(v, (MT, T))

    ltx = jnp.maximum(row(ax1), col(bx1))
    lty = jnp.maximum(row(ay1), col(by1))
    rbx = jnp.minimum(row(ax2), col(bx2))
    rby = jnp.minimum(row(ay2), col(by2))
    ww = jnp.maximum(rbx - ltx, 0.0)
    hh = jnp.maximum(rby - lty, 0.0)
    inter = ww * hh
    uni = (row(area_a) + col(area_b)) - inter
    iou = inter / uni
    iou = jnp.where(tcol < M, iou, -1.0)

    rm = jnp.max(iou, axis=0)
    ra = jnp.min(jnp.where(iou == rm[None, :], tcol, BIG), axis=0)
    rm_ref[0, 0, 0, :] = rm
    ra_ref[0, 0, 0, :] = ra

    cmt = jnp.max(iou, axis=1, keepdims=True)
    cat = jnp.min(jnp.where(iou == cmt, aid, BIG), axis=1, keepdims=True)
    prev_cm = jnp.where(k == 0, -2.0, cm_ref[0, :, :])
    prev_ca = jnp.where(k == 0, BIG, ca_ref[0, :, :])
    upd = cmt > prev_cm
    eq = cmt == prev_cm
    cm_ref[0, :, :] = jnp.maximum(prev_cm, cmt)
    ca_ref[0, :, :] = jnp.where(upd, cat,
                                jnp.where(eq, jnp.minimum(prev_ca, cat),
                                          prev_ca))


def _sc_body(rm_hbm, ra_hbm, cm_hbm, ca_hbm, bb_hbm, lab_hbm,
             sc_hbm, obb_hbm, *scr):
    sets = (scr[0:9], scr[9:18])
    m_den, scb, bb2, sem = scr[18:]
    c = lax.axis_index("c")
    s = lax.axis_index("s")
    base = s * CH
    iota = lax.iota(i32, L)

    def fire(i, refs):
        r_iou, r_arg, m_iou, m_id, bx1, by1, bx2, by2, labv = refs
        b = c * (B // NC) + i
        return [
            pltpu.async_copy(rm_hbm.at[b, pl.ds(base, CH)], r_iou, sem),
            pltpu.async_copy(ra_hbm.at[b, pl.ds(base, CH)], r_arg, sem),
            pltpu.async_copy(cm_hbm.at[b, :], m_iou, sem),
            pltpu.async_copy(ca_hbm.at[b, :], m_id, sem),
            pltpu.async_copy(bb_hbm.at[b, 0, :], bx1, sem),
            pltpu.async_copy(bb_hbm.at[b, 1, :], by1, sem),
            pltpu.async_copy(bb_hbm.at[b, 2, :], bx2, sem),
            pltpu.async_copy(bb_hbm.at[b, 3, :], by2, sem),
            pltpu.async_copy(lab_hbm.at[b, :], labv, sem),
        ]

    def compute(i, refs):
        r_iou, r_arg, m_iou, m_id, bx1, by1, bx2, by2, labv = refs
        b = c * (B // NC) + i

        def den_body(tg, _):
            sl = pl.ds(tg * L, L)
            m_den[sl] = jnp.maximum(m_iou[sl], IOU_T)
            return 0

        lax.fori_loop(0, MP // L, den_body, 0)

        def ow_body(t, _):
            tsl16 = pl.ds(t, L)
            ai = m_id[tsl16][0]
            off = ai - base
            inr = (off >= 0) & (off < CH)
            offc = jnp.clip(off, 0, CH - 1)
            mk = lax.bitwise_and(iota == 0, jnp.broadcast_to(inr, (L,)))
            iv = jnp.broadcast_to(offc, (L,)).astype(i32)
            plsc.store_scatter(r_iou, [iv],
                               jnp.broadcast_to(m_iou[tsl16][0], (L,)),
                               mask=mk)
            plsc.store_scatter(r_arg, [iv],
                               jnp.broadcast_to(t, (L,)).astype(i32), mask=mk)
            return 0

        lax.fori_loop(0, M, ow_body, 0)

        def fin_body(g, _):
            gs = g * L
            sl = pl.ds(gs, L)
            mi = r_iou[sl]
            bidx = r_arg[sl]
            den = plsc.load_gather(m_den, [bidx])
            lv = plsc.load_gather(labv, [bidx])
            mia = jnp.where(mi < IOU_T * 0.5, 0.0, mi)
            scv = mia / den
            scv = jnp.where(lv <= 0, jnp.zeros((L,), f32), scv)
            scb[sl] = scv
            rows4 = (jnp.broadcast_to(gs, (L,)).astype(i32) + iota) * 4
            for j, ref in enumerate((bx1, by1, bx2, by2)):
                bbj = plsc.load_gather(ref, [bidx])
                plsc.store_scatter(bb2, [rows4 + j], bbj)
            return 0

        lax.fori_loop(0, G, fin_body, 0)

        o1 = pltpu.async_copy(scb, sc_hbm.at[b, pl.ds(base, CH)], sem)
        o2 = pltpu.async_copy(bb2, obb_hbm.at[b, pl.ds(base * 4, CH * 4)],
                              sem)
        return o1, o2

    nimg = B // NC
    pend = fire(0, sets[0])
    outs = None
    for i in range(nimg):
        for cp in pend:
            cp.wait()
        if i + 1 < nimg:
            pend = fire(i + 1, sets[(i + 1) % 2])
        if outs is not None:
            outs[0].wait()
            outs[1].wait()
        outs = compute(i, sets[i % 2])
    outs[0].wait()
    outs[1].wait()


@jax.jit
def kernel(labels, bboxes, anchors):
    labp = jnp.pad(labels.astype(i32), ((0, 0), (0, MP - M)))
    bb_soa = jnp.transpose(jnp.pad(bboxes, ((0, 0), (0, MP - M), (0, 0))),
                           (0, 2, 1))
    bbT = bb_soa[:, :, :MT, None]
    anp = jnp.transpose(jnp.pad(anchors, ((0, AP - A), (0, 0))), (1, 0))

    rm, ra, cm, ca = pl.pallas_call(
        _tc_body,
        grid=(B, NT),
        in_specs=[
            pl.BlockSpec((4, T), lambda b, k: (0, k)),
            pl.BlockSpec((1, 4, MT, 1), lambda b, k: (b, 0, 0, 0)),
        ],
        out_specs=[
            pl.BlockSpec((1, 1, 1, T), lambda b, k: (b, k, 0, 0)),
            pl.BlockSpec((1, 1, 1, T), lambda b, k: (b, k, 0, 0)),
            pl.BlockSpec((1, MT, 1), lambda b, k: (b, 0, 0)),
            pl.BlockSpec((1, MT, 1), lambda b, k: (b, 0, 0)),
        ],
        out_shape=[
            jax.ShapeDtypeStruct((B, NT, 1, T), f32),
            jax.ShapeDtypeStruct((B, NT, 1, T), i32),
            jax.ShapeDtypeStruct((B, MT, 1), f32),
            jax.ShapeDtypeStruct((B, MT, 1), i32),
        ],
    )(anp, bbT)

    rm = rm.reshape(B, AP)
    ra = ra.reshape(B, AP)
    cm = jnp.pad(cm.reshape(B, MT), ((0, 0), (0, MP - MT)))
    ca = jnp.pad(ca.reshape(B, MT), ((0, 0), (0, MP - MT)))

    mesh = plsc.VectorSubcoreMesh(core_axis_name="c", subcore_axis_name="s",
                                  num_cores=NC, num_subcores=NS)
    run = pl.kernel(
        _sc_body,
        out_type=[jax.ShapeDtypeStruct((B, AP), f32),
                  jax.ShapeDtypeStruct((B, AP * 4), f32)],
        mesh=mesh,
        compiler_params=pltpu.CompilerParams(needs_layout_passes=False),
        scratch_types=[
            pltpu.VMEM((CH,), f32),
            pltpu.VMEM((CH,), i32),
            pltpu.VMEM((MP,), f32),
            pltpu.VMEM((MP,), i32),
            pltpu.VMEM((MP,), f32),
            pltpu.VMEM((MP,), f32),
            pltpu.VMEM((MP,), f32),
            pltpu.VMEM((MP,), f32),
            pltpu.VMEM((MP,), i32),
            pltpu.VMEM((CH,), f32),
            pltpu.VMEM((CH,), i32),
            pltpu.VMEM((MP,), f32),
            pltpu.VMEM((MP,), i32),
            pltpu.VMEM((MP,), f32),
            pltpu.VMEM((MP,), f32),
            pltpu.VMEM((MP,), f32),
            pltpu.VMEM((MP,), f32),
            pltpu.VMEM((MP,), i32),
            pltpu.VMEM((MP,), f32),
            pltpu.VMEM((CH,), f32),
            pltpu.VMEM((CH * 4,), f32),
            pltpu.SemaphoreType.DMA,
        ],
    )
    scores, obb = run(rm, ra, cm, ca, bb_soa, labp)
    return scores[:, :A], obb.reshape(B, AP, 4)[:, :A, :]
